# baseline (speedup 1.0000x reference)
import functools

import jax
import jax.numpy as jnp
from jax import lax
from jax.experimental import pallas as pl
from jax.experimental.pallas import tpu as pltpu

_ALPHA = 0.5
_BETA = 0.5
_SMOOTH = 1.0
_TOP_PERCENT = 0.05
_TAU = 0.1
_BCE_WEIGHT = 0.5
_FOCAL_WEIGHT = 0.5
_EPS = 1e-12


def _u16_to_f32(u16):
    b16 = jnp.where((u16 & 0x8000) != 0, u16 & 0x7FFF, (~u16) & 0xFFFF)
    return lax.bitcast_convert_type(jnp.left_shift(b16, 16), jnp.float32)


def _loss_kernel(k, logits_ref, targets_ref, out_ref):
    x = logits_ref[...]
    t = targets_ref[...].astype(jnp.float32)
    xb = x.astype(jnp.bfloat16)

    def body(i, prefix):
        j = 15 - i
        cand_u = prefix | jnp.left_shift(jnp.int32(1), j)
        cand = _u16_to_f32(cand_u).astype(jnp.bfloat16)
        c = jnp.sum(jnp.sum((xb >= cand).astype(jnp.bfloat16), axis=0)
                    .astype(jnp.float32))
        return jnp.where(c >= k, cand_u, prefix)

    p_u = lax.fori_loop(0, 16, body, jnp.int32(0))
    x_k = _u16_to_f32(p_u)
    q = 1.0 / (1.0 + jnp.exp(-x_k))

    e = jnp.exp(-x)
    p = 1.0 / (1.0 + e)
    m = 1.0 / (1.0 + jnp.exp((q - p) / _TAU))
    log_p = jnp.maximum(-jnp.log1p(e), -27.631021)
    log_1mp = jnp.maximum(log_p - x, -27.631021)
    bce = -(t * log_p + (1.0 - t) * log_1mp)
    one_minus_pt = jnp.where(t == 1.0, 1.0 - p, p)
    focal = one_minus_pt * one_minus_pt * bce

    def rsum(v):
        return jnp.sum(jnp.sum(v, axis=1))

    sum_t = rsum(t)
    sum_m = rsum(m)
    sum_mt = rsum(m * t)
    sum_bce = rsum(bce)
    sum_focal = rsum(focal)

    n = jnp.float32(x.size)
    tp = sum_mt
    fp = sum_m - sum_mt
    fn = sum_t - sum_mt
    tversky = (tp + _SMOOTH) / (tp + _ALPHA * fp + _BETA * fn + _SMOOTH)
    loss = (1.0 - tversky) + _BCE_WEIGHT * sum_bce / n + _FOCAL_WEIGHT * sum_focal / n
    out_ref[0, 0] = loss


def kernel(logits, targets, metadata=0):
    n = logits.size
    k = max(1, int(_TOP_PERCENT * n))
    out = pl.pallas_call(
        functools.partial(_loss_kernel, k),
        out_shape=jax.ShapeDtypeStruct((1, 1), jnp.float32),
        out_specs=pl.BlockSpec(memory_space=pltpu.SMEM),
    )(logits, targets)
    return out[0, 0]


# speedup vs baseline: 1.2291x; 1.2291x over previous
import functools

import jax
import jax.numpy as jnp
from jax import lax
from jax.experimental import pallas as pl
from jax.experimental.pallas import tpu as pltpu

_ALPHA = 0.5
_BETA = 0.5
_SMOOTH = 1.0
_TOP_PERCENT = 0.05
_TAU = 0.1
_BCE_WEIGHT = 0.5
_FOCAL_WEIGHT = 0.5
_EPS = 1e-12


def _u16_to_f32(u16):
    b16 = jnp.where((u16 & 0x8000) != 0, u16 & 0x7FFF, (~u16) & 0xFFFF)
    return lax.bitcast_convert_type(jnp.left_shift(b16, 16), jnp.float32)


def _loss_kernel(k, logits_ref, targets_ref, out_ref):
    x = logits_ref[...]
    t = targets_ref[...].astype(jnp.float32)
    xb = x.astype(jnp.bfloat16)

    def body(i, prefix):
        j = 15 - i
        cand_u = prefix | jnp.left_shift(jnp.int32(1), j)
        cand = _u16_to_f32(cand_u).astype(jnp.bfloat16)
        c = jnp.sum(jnp.sum((xb >= cand).astype(jnp.float32), axis=1))
        return jnp.where(c >= k, cand_u, prefix)

    p_u = lax.fori_loop(0, 16, body, jnp.int32(0))
    x_k = _u16_to_f32(p_u)
    q = 1.0 / (1.0 + jnp.exp(-x_k))

    e = jnp.exp(-x)
    p = 1.0 / (1.0 + e)
    m = 1.0 / (1.0 + jnp.exp((q - p) / _TAU))
    log_p = jnp.maximum(-jnp.log1p(e), -27.631021)
    log_1mp = jnp.maximum(log_p - x, -27.631021)
    bce = -(t * log_p + (1.0 - t) * log_1mp)
    one_minus_pt = jnp.where(t == 1.0, 1.0 - p, p)
    focal = one_minus_pt * one_minus_pt * bce

    def rsum(v):
        return jnp.sum(jnp.sum(v, axis=1))

    sum_t = rsum(t)
    sum_m = rsum(m)
    sum_mt = rsum(m * t)
    sum_bce = rsum(bce)
    sum_focal = rsum(focal)

    n = jnp.float32(x.size)
    tp = sum_mt
    fp = sum_m - sum_mt
    fn = sum_t - sum_mt
    tversky = (tp + _SMOOTH) / (tp + _ALPHA * fp + _BETA * fn + _SMOOTH)
    loss = (1.0 - tversky) + _BCE_WEIGHT * sum_bce / n + _FOCAL_WEIGHT * sum_focal / n
    out_ref[0, 0] = loss


def kernel(logits, targets, metadata=0):
    n = logits.size
    k = max(1, int(_TOP_PERCENT * n))
    out = pl.pallas_call(
        functools.partial(_loss_kernel, k),
        out_shape=jax.ShapeDtypeStruct((1, 1), jnp.float32),
        out_specs=pl.BlockSpec(memory_space=pltpu.SMEM),
    )(logits, targets)
    return out[0, 0]


# speedup vs baseline: 1.3495x; 1.0979x over previous
import functools

import jax
import jax.numpy as jnp
from jax import lax
from jax.experimental import pallas as pl
from jax.experimental.pallas import tpu as pltpu

_ALPHA = 0.5
_BETA = 0.5
_SMOOTH = 1.0
_TOP_PERCENT = 0.05
_TAU = 0.1
_BCE_WEIGHT = 0.5
_FOCAL_WEIGHT = 0.5
_EPS = 1e-12


def _u16_to_f32(u16):
    b16 = jnp.where((u16 & 0x8000) != 0, u16 & 0x7FFF, (~u16) & 0xFFFF)
    return lax.bitcast_convert_type(jnp.left_shift(b16, 16), jnp.float32)


def _loss_kernel(k, logits_ref, targets_ref, out_ref):
    x = logits_ref[...]
    t = targets_ref[...].astype(jnp.float32)
    xb = x.astype(jnp.bfloat16)

    def body(i, prefix):
        j = 15 - i
        cand_u = prefix | jnp.left_shift(jnp.int32(1), j)
        cand = _u16_to_f32(cand_u).astype(jnp.bfloat16)
        c = jnp.sum(jnp.sum((xb >= cand).astype(jnp.float32), axis=1))
        return jnp.where(c >= k, cand_u, prefix)

    p_u = jnp.int32(0)
    for i in range(16):
        p_u = body(i, p_u)
    x_k = _u16_to_f32(p_u)
    q = 1.0 / (1.0 + jnp.exp(-x_k))

    p = 1.0 / (1.0 + jnp.exp(-x))
    m = 1.0 / (1.0 + jnp.exp((q - p) / _TAU))
    p_c = jnp.clip(p, _EPS, 1.0 - _EPS)
    bce = -(t * jnp.log(p_c) + (1.0 - t) * jnp.log(1.0 - p_c))
    one_minus_pt = jnp.where(t == 1.0, 1.0 - p, p)
    focal = one_minus_pt * one_minus_pt * bce

    def rsum(v):
        return jnp.sum(jnp.sum(v, axis=1))

    sum_t = rsum(t)
    sum_m = rsum(m)
    sum_mt = rsum(m * t)
    sum_bce = rsum(bce)
    sum_focal = rsum(focal)

    n = jnp.float32(x.size)
    tp = sum_mt
    fp = sum_m - sum_mt
    fn = sum_t - sum_mt
    tversky = (tp + _SMOOTH) / (tp + _ALPHA * fp + _BETA * fn + _SMOOTH)
    loss = (1.0 - tversky) + _BCE_WEIGHT * sum_bce / n + _FOCAL_WEIGHT * sum_focal / n
    out_ref[0, 0] = loss


def kernel(logits, targets, metadata=0):
    n = logits.size
    k = max(1, int(_TOP_PERCENT * n))
    out = pl.pallas_call(
        functools.partial(_loss_kernel, k),
        out_shape=jax.ShapeDtypeStruct((1, 1), jnp.float32),
        out_specs=pl.BlockSpec(memory_space=pltpu.SMEM),
    )(logits, targets)
    return out[0, 0]


# speedup vs baseline: 1.3820x; 1.0241x over previous
import functools

import jax
import jax.numpy as jnp
from jax import lax
from jax.experimental import pallas as pl
from jax.experimental.pallas import tpu as pltpu

_ALPHA = 0.5
_BETA = 0.5
_SMOOTH = 1.0
_TOP_PERCENT = 0.05
_TAU = 0.1
_BCE_WEIGHT = 0.5
_FOCAL_WEIGHT = 0.5
_EPS = 1e-12


def _u16_to_f32(u16):
    b16 = jnp.where((u16 & 0x8000) != 0, u16 & 0x7FFF, (~u16) & 0xFFFF)
    return lax.bitcast_convert_type(jnp.left_shift(b16, 16), jnp.float32)


def _loss_kernel(k, logits_ref, targets_ref, out_ref, t_vmem, t_sem):
    cp = pltpu.make_async_copy(targets_ref, t_vmem, t_sem)
    cp.start()
    x = logits_ref[...]
    xb = x.astype(jnp.bfloat16)

    def body(i, prefix):
        j = 15 - i
        cand_u = prefix | jnp.left_shift(jnp.int32(1), j)
        cand = _u16_to_f32(cand_u).astype(jnp.bfloat16)
        c = jnp.sum(jnp.sum((xb >= cand).astype(jnp.float32), axis=1))
        return jnp.where(c >= k, cand_u, prefix)

    p_u = jnp.int32(0)
    for i in range(16):
        p_u = body(i, p_u)
    x_k = _u16_to_f32(p_u)
    cp.wait()
    t = t_vmem[...].astype(jnp.float32)
    q = 1.0 / (1.0 + jnp.exp(-x_k))

    p = 1.0 / (1.0 + jnp.exp(-x))
    m = 1.0 / (1.0 + jnp.exp((q - p) / _TAU))
    p_c = jnp.clip(p, _EPS, 1.0 - _EPS)
    bce = -(t * jnp.log(p_c) + (1.0 - t) * jnp.log(1.0 - p_c))
    one_minus_pt = jnp.where(t == 1.0, 1.0 - p, p)
    focal = one_minus_pt * one_minus_pt * bce

    def rsum(v):
        return jnp.sum(jnp.sum(v, axis=1))

    sum_t = rsum(t)
    sum_m = rsum(m)
    sum_mt = rsum(m * t)
    sum_bce = rsum(bce)
    sum_focal = rsum(focal)

    n = jnp.float32(x.size)
    tp = sum_mt
    fp = sum_m - sum_mt
    fn = sum_t - sum_mt
    tversky = (tp + _SMOOTH) / (tp + _ALPHA * fp + _BETA * fn + _SMOOTH)
    loss = (1.0 - tversky) + _BCE_WEIGHT * sum_bce / n + _FOCAL_WEIGHT * sum_focal / n
    out_ref[0, 0] = loss


def kernel(logits, targets, metadata=0):
    n = logits.size
    k = max(1, int(_TOP_PERCENT * n))
    out = pl.pallas_call(
        functools.partial(_loss_kernel, k),
        out_shape=jax.ShapeDtypeStruct((1, 1), jnp.float32),
        in_specs=[
            pl.BlockSpec(memory_space=pltpu.MemorySpace.VMEM),
            pl.BlockSpec(memory_space=pltpu.MemorySpace.HBM),
        ],
        out_specs=pl.BlockSpec(memory_space=pltpu.SMEM),
        scratch_shapes=[
            pltpu.VMEM((64, 8192), jnp.int32),
            pltpu.SemaphoreType.DMA,
        ],
    )(logits, targets)
    return out[0, 0]


# speedup vs baseline: 1.4349x; 1.0383x over previous
import functools

import jax
import jax.numpy as jnp
from jax import lax
from jax.experimental import pallas as pl
from jax.experimental.pallas import tpu as pltpu

_ALPHA = 0.5
_BETA = 0.5
_SMOOTH = 1.0
_TOP_PERCENT = 0.05
_TAU = 0.1
_BCE_WEIGHT = 0.5
_FOCAL_WEIGHT = 0.5
_EPS = 1e-12


def _u16_to_f32(u16):
    b16 = jnp.where((u16 & 0x8000) != 0, u16 & 0x7FFF, (~u16) & 0xFFFF)
    return lax.bitcast_convert_type(jnp.left_shift(b16, 16), jnp.float32)


def _loss_kernel(k, logits_ref, targets_ref, out_ref, t_vmem, t_sem):
    cp = pltpu.make_async_copy(targets_ref, t_vmem, t_sem)
    cp.start()
    x = logits_ref[...]
    xb = x.astype(jnp.bfloat16)

    def body(i, prefix):
        j = 15 - i
        cand_u = prefix | jnp.left_shift(jnp.int32(1), j)
        cand = _u16_to_f32(cand_u).astype(jnp.bfloat16)
        c = jnp.sum(jnp.sum((xb >= cand).astype(jnp.float32), axis=1))
        return jnp.where(c >= k, cand_u, prefix)

    p_u = jnp.int32(0)
    for i in range(15):
        p_u = body(i, p_u)
    x_k = _u16_to_f32(p_u)
    cp.wait()
    t = t_vmem[...].astype(jnp.float32)
    q = 1.0 / (1.0 + jnp.exp(-x_k))

    p = 1.0 / (1.0 + jnp.exp(-x))
    m = 1.0 / (1.0 + jnp.exp((q - p) / _TAU))
    p_c = jnp.clip(p, _EPS, 1.0 - _EPS)
    bce = -(t * jnp.log(p_c) + (1.0 - t) * jnp.log(1.0 - p_c))
    one_minus_pt = jnp.where(t == 1.0, 1.0 - p, p)
    focal = one_minus_pt * one_minus_pt * bce

    def rsum(v):
        return jnp.sum(jnp.sum(v, axis=1))

    sum_t = rsum(t)
    sum_m = rsum(m)
    sum_mt = rsum(m * t)
    sum_bce = rsum(bce)
    sum_focal = rsum(focal)

    n = jnp.float32(x.size)
    tp = sum_mt
    fp = sum_m - sum_mt
    fn = sum_t - sum_mt
    tversky = (tp + _SMOOTH) / (tp + _ALPHA * fp + _BETA * fn + _SMOOTH)
    loss = (1.0 - tversky) + _BCE_WEIGHT * sum_bce / n + _FOCAL_WEIGHT * sum_focal / n
    out_ref[0, 0] = loss


def kernel(logits, targets, metadata=0):
    n = logits.size
    k = max(1, int(_TOP_PERCENT * n))
    out = pl.pallas_call(
        functools.partial(_loss_kernel, k),
        out_shape=jax.ShapeDtypeStruct((1, 1), jnp.float32),
        in_specs=[
            pl.BlockSpec(memory_space=pltpu.MemorySpace.VMEM),
            pl.BlockSpec(memory_space=pltpu.MemorySpace.HBM),
        ],
        out_specs=pl.BlockSpec(memory_space=pltpu.SMEM),
        scratch_shapes=[
            pltpu.VMEM((64, 8192), jnp.int32),
            pltpu.SemaphoreType.DMA,
        ],
    )(logits, targets)
    return out[0, 0]


# speedup vs baseline: 1.5169x; 1.0572x over previous
import functools

import jax
import jax.numpy as jnp
from jax import lax
from jax.experimental import pallas as pl
from jax.experimental.pallas import tpu as pltpu

_ALPHA = 0.5
_BETA = 0.5
_SMOOTH = 1.0
_TOP_PERCENT = 0.05
_TAU = 0.1
_BCE_WEIGHT = 0.5
_FOCAL_WEIGHT = 0.5
_EPS = 1e-12


def _u16_to_f32(u16):
    b16 = jnp.where((u16 & 0x8000) != 0, u16 & 0x7FFF, (~u16) & 0xFFFF)
    return lax.bitcast_convert_type(jnp.left_shift(b16, 16), jnp.float32)


def _loss_kernel(k, logits_ref, targets_ref, out_ref, t_vmem, t_sem):
    cp = pltpu.make_async_copy(targets_ref, t_vmem, t_sem)
    cp.start()
    x = logits_ref[...]
    xb = x.astype(jnp.bfloat16)

    def body(i, prefix):
        j = 15 - i
        cand_u = prefix | jnp.left_shift(jnp.int32(1), j)
        cand = _u16_to_f32(cand_u).astype(jnp.bfloat16)
        c = jnp.sum(jnp.sum((xb >= cand).astype(jnp.float32), axis=1))
        return jnp.where(c >= k, cand_u, prefix)

    p_parts, lpc_parts, l1pc_parts = [], [], []

    def eup_chunk(i):
        xi = x[4 * i:4 * i + 4]
        pi = 1.0 / (1.0 + jnp.exp(-xi))
        pci = jnp.clip(pi, _EPS, 1.0 - _EPS)
        p_parts.append(pi)
        lpc_parts.append(jnp.log(pci))
        l1pc_parts.append(jnp.log(1.0 - pci))

    p_u = jnp.int32(0)
    for i in range(15):
        p_u = body(i, p_u)
        eup_chunk(i)
    eup_chunk(15)
    x_k = _u16_to_f32(p_u)
    cp.wait()
    t = t_vmem[...].astype(jnp.float32)
    q = 1.0 / (1.0 + jnp.exp(-x_k))

    p = jnp.concatenate(p_parts, axis=0)
    log_pc = jnp.concatenate(lpc_parts, axis=0)
    log_1pc = jnp.concatenate(l1pc_parts, axis=0)
    m = 1.0 / (1.0 + jnp.exp((q - p) / _TAU))
    bce = -(t * log_pc + (1.0 - t) * log_1pc)
    one_minus_pt = jnp.where(t == 1.0, 1.0 - p, p)
    focal = one_minus_pt * one_minus_pt * bce

    def rsum(v):
        return jnp.sum(jnp.sum(v, axis=1))

    sum_t = rsum(t)
    sum_m = rsum(m)
    sum_mt = rsum(m * t)
    sum_bce = rsum(bce)
    sum_focal = rsum(focal)

    n = jnp.float32(x.size)
    tp = sum_mt
    fp = sum_m - sum_mt
    fn = sum_t - sum_mt
    tversky = (tp + _SMOOTH) / (tp + _ALPHA * fp + _BETA * fn + _SMOOTH)
    loss = (1.0 - tversky) + _BCE_WEIGHT * sum_bce / n + _FOCAL_WEIGHT * sum_focal / n
    out_ref[0, 0] = loss


def kernel(logits, targets, metadata=0):
    n = logits.size
    k = max(1, int(_TOP_PERCENT * n))
    out = pl.pallas_call(
        functools.partial(_loss_kernel, k),
        out_shape=jax.ShapeDtypeStruct((1, 1), jnp.float32),
        in_specs=[
            pl.BlockSpec(memory_space=pltpu.MemorySpace.VMEM),
            pl.BlockSpec(memory_space=pltpu.MemorySpace.HBM),
        ],
        out_specs=pl.BlockSpec(memory_space=pltpu.SMEM),
        scratch_shapes=[
            pltpu.VMEM((64, 8192), jnp.int32),
            pltpu.SemaphoreType.DMA,
        ],
    )(logits, targets)
    return out[0, 0]
